# Initial kernel scaffold; baseline (speedup 1.0000x reference)
import functools

import jax
import jax.numpy as jnp
from jax import lax
from jax.experimental import pallas as pl
from jax.experimental.pallas import tpu as pltpu
from jax.experimental.pallas import tpu_sc as plsc

_MARGIN = 1.0
_L = 16


def _triplet_kernel(T, B, D, NW, TPW):
    mesh = plsc.VectorSubcoreMesh(core_axis_name="c", subcore_axis_name="s")

    @functools.partial(
        pl.kernel,
        mesh=mesh,
        out_type=jax.ShapeDtypeStruct((NW, _L), jnp.float32),
        scratch_types=[
            pltpu.VMEM((TPW,), jnp.int32),
            pltpu.VMEM((TPW,), jnp.int32),
            pltpu.VMEM((TPW,), jnp.int32),
            pltpu.VMEM((TPW, D), jnp.float32),
            pltpu.VMEM((TPW, D), jnp.float32),
            pltpu.VMEM((TPW, D), jnp.float32),
            pltpu.VMEM((_L,), jnp.float32),
            pltpu.SemaphoreType.DMA,
        ],
    )
    def k(emb_hbm, trip_hbm, out_hbm, ia_v, ip_v, in_v, a_v, p_v, n_v,
          out_v, sem):
        wid = lax.axis_index("s") * 2 + lax.axis_index("c")
        base = wid * TPW

        pltpu.sync_copy(trip_hbm.at[0, pl.ds(base, TPW)], ia_v)
        pltpu.sync_copy(trip_hbm.at[1, pl.ds(base, TPW)], ip_v)
        pltpu.sync_copy(trip_hbm.at[2, pl.ds(base, TPW)], in_v)

        ca = pltpu.make_async_copy(emb_hbm.at[ia_v], a_v, sem)
        cp = pltpu.make_async_copy(emb_hbm.at[ip_v], p_v, sem)
        cn = pltpu.make_async_copy(emb_hbm.at[in_v], n_v, sem)
        ca.start()
        cp.start()
        cn.start()
        ca.wait()
        cp.wait()
        cn.wait()

        def body(t, acc):
            lane = jnp.zeros((_L,), jnp.float32)
            for c in range(D // _L):
                sl = pl.ds(c * _L, _L)
                a = a_v[t, sl]
                p = p_v[t, sl]
                n = n_v[t, sl]
                ap = a - p
                an = a - n
                lane = lane + (ap * ap - an * an)
            s = jnp.sum(lane) + _MARGIN
            return acc + jnp.maximum(s, 0.0)

        total = lax.fori_loop(0, TPW, body, jnp.float32(0.0))
        out_v[...] = jnp.full((_L,), total, jnp.float32)
        pltpu.sync_copy(out_v, out_hbm.at[wid])

    return k


def kernel(embeddings, target, triplets):
    del target
    T = triplets.shape[0]
    B, D = embeddings.shape
    NW = 32
    TPW = T // NW
    trip_t = triplets.T.astype(jnp.int32)
    partials = _triplet_kernel(T, B, D, NW, TPW)(embeddings, trip_t)
    return (jnp.sum(partials[:, 0]) / T, T)


# speedup vs baseline: 1.6740x; 1.6740x over previous
import functools

import jax
import jax.numpy as jnp
from jax import lax
from jax.experimental import pallas as pl
from jax.experimental.pallas import tpu as pltpu
from jax.experimental.pallas import tpu_sc as plsc

_MARGIN = 1.0
_L = 16


def _triplet_kernel(T, B, D, NW, TPW):
    mesh = plsc.VectorSubcoreMesh(core_axis_name="c", subcore_axis_name="s")

    @functools.partial(
        pl.kernel,
        mesh=mesh,
        out_type=jax.ShapeDtypeStruct((NW, _L), jnp.float32),
        compiler_params=pltpu.CompilerParams(use_tc_tiling_on_sc=False),
        scratch_types=[
            pltpu.VMEM((TPW,), jnp.int32),
            pltpu.VMEM((TPW,), jnp.int32),
            pltpu.VMEM((TPW,), jnp.int32),
            pltpu.VMEM((TPW, D), jnp.float32),
            pltpu.VMEM((TPW, D), jnp.float32),
            pltpu.VMEM((TPW, D), jnp.float32),
            pltpu.VMEM((_L,), jnp.float32),
            pltpu.SemaphoreType.DMA,
        ],
    )
    def k(emb_hbm, ia_hbm, ip_hbm, in_hbm, out_hbm, ia_v, ip_v, in_v,
          a_v, p_v, n_v, out_v, sem):
        wid = lax.axis_index("s") * 2 + lax.axis_index("c")
        base = wid * TPW

        pltpu.sync_copy(ia_hbm.at[pl.ds(base, TPW)], ia_v)
        pltpu.sync_copy(ip_hbm.at[pl.ds(base, TPW)], ip_v)
        pltpu.sync_copy(in_hbm.at[pl.ds(base, TPW)], in_v)

        ca = pltpu.make_async_copy(emb_hbm.at[ia_v], a_v, sem)
        cp = pltpu.make_async_copy(emb_hbm.at[ip_v], p_v, sem)
        cn = pltpu.make_async_copy(emb_hbm.at[in_v], n_v, sem)
        ca.start()
        cp.start()
        cn.start()
        ca.wait()
        cp.wait()
        cn.wait()

        lanes = lax.iota(jnp.int32, _L)
        perms = [lanes ^ sh for sh in (8, 4, 2, 1)]

        def body(t, acc):
            lane = jnp.zeros((_L,), jnp.float32)
            for c in range(D // _L):
                sl = pl.ds(c * _L, _L)
                a = a_v[t, sl]
                p = p_v[t, sl]
                n = n_v[t, sl]
                ap = a - p
                an = a - n
                lane = lane + (ap * ap - an * an)
            dnums = lax.GatherDimensionNumbers(
                offset_dims=(), collapsed_slice_dims=(0,),
                start_index_map=(0,))
            for perm in perms:
                lane = lane + lax.gather(
                    lane, perm[:, None], dnums, (1,),
                    mode=lax.GatherScatterMode.PROMISE_IN_BOUNDS)
            return acc + jnp.maximum(lane + _MARGIN, 0.0)

        total = lax.fori_loop(0, TPW, body, jnp.zeros((_L,), jnp.float32))
        out_v[...] = total
        pltpu.sync_copy(out_v, out_hbm.at[wid])

    return k


def kernel(embeddings, target, triplets):
    del target
    T = triplets.shape[0]
    B, D = embeddings.shape
    NW = 32
    TPW = T // NW
    ia = triplets[:, 0]
    ip = triplets[:, 1]
    inn = triplets[:, 2]
    partials = _triplet_kernel(T, B, D, NW, TPW)(embeddings, ia, ip, inn)
    return (jnp.sum(partials[:, 0]) / T, T)
